# Initial kernel scaffold; baseline (speedup 1.0000x reference)
import functools
import jax
import jax.numpy as jnp
from jax.experimental import pallas as pl
from jax.experimental.pallas import tpu as pltpu

N_TOK = 4096
D_MODEL = 2048
N_EXP = 8
TOPK = 2


def _gating_body(x_ref, gw_ref, gb_ref, idx_ref, w_ref, comb_ref):
    x = x_ref[...]
    logits = jax.lax.dot_general(
        x, gw_ref[...], (((1,), (1,)), ((), ())),
        preferred_element_type=jnp.float32) + gb_ref[...]
    e_iota = jax.lax.broadcasted_iota(jnp.int32, logits.shape, 1)
    m1 = jnp.max(logits, axis=1, keepdims=True)
    i1 = jnp.min(jnp.where(logits == m1, e_iota, N_EXP), axis=1, keepdims=True)
    l2 = jnp.where(e_iota == i1, -jnp.inf, logits)
    m2 = jnp.max(l2, axis=1, keepdims=True)
    i2 = jnp.min(jnp.where(l2 == m2, e_iota, N_EXP), axis=1, keepdims=True)
    w0 = 1.0 / (1.0 + jnp.exp(m2 - m1))
    w1 = 1.0 - w0
    idx_ref[...] = jnp.concatenate([i1, i2], axis=1)
    w_ref[...] = jnp.concatenate([w0, w1], axis=1)
    comb_ref[...] = (jnp.where(e_iota == i1, w0, 0.0)
                     + jnp.where(e_iota == i2, w1, 0.0))


def _gating(x, gate_w, gate_b):
    blk = 512
    grid = N_TOK // blk
    return pl.pallas_call(
        _gating_body,
        grid=(grid,),
        in_specs=[
            pl.BlockSpec((blk, D_MODEL), lambda t: (t, 0)),
            pl.BlockSpec((N_EXP, D_MODEL), lambda t: (0, 0)),
            pl.BlockSpec((1, N_EXP), lambda t: (0, 0)),
        ],
        out_specs=[
            pl.BlockSpec((blk, TOPK), lambda t: (t, 0)),
            pl.BlockSpec((blk, TOPK), lambda t: (t, 0)),
            pl.BlockSpec((blk, N_EXP), lambda t: (t, 0)),
        ],
        out_shape=[
            jax.ShapeDtypeStruct((N_TOK, TOPK), jnp.int32),
            jax.ShapeDtypeStruct((N_TOK, TOPK), jnp.float32),
            jax.ShapeDtypeStruct((N_TOK, N_EXP), jnp.float32),
        ],
    )(x, gate_w, gate_b.reshape(1, N_EXP))


def _dense_body(x_ref, w_ref, b_ref, comb_ref, out_ref):
    e = pl.program_id(1)
    y = jax.lax.dot_general(
        x_ref[...], w_ref[0], (((1,), (1,)), ((), ())),
        preferred_element_type=jnp.float32)
    contrib = (y + b_ref[...]) * comb_ref[...]

    @pl.when(e == 0)
    def _():
        out_ref[...] = contrib

    @pl.when(e > 0)
    def _():
        out_ref[...] = out_ref[...] + contrib


def _dense_experts(x, expert_w, expert_b, comb):
    tblk = 1024
    grid = (N_TOK // tblk, N_EXP)
    return pl.pallas_call(
        _dense_body,
        grid=grid,
        in_specs=[
            pl.BlockSpec((tblk, D_MODEL), lambda t, e: (t, 0)),
            pl.BlockSpec((1, D_MODEL, D_MODEL), lambda t, e: (e, 0, 0)),
            pl.BlockSpec((1, D_MODEL), lambda t, e: (e, 0)),
            pl.BlockSpec((tblk, 1), lambda t, e: (t, e)),
        ],
        out_specs=pl.BlockSpec((tblk, D_MODEL), lambda t, e: (t, 0)),
        out_shape=jax.ShapeDtypeStruct((N_TOK, D_MODEL), jnp.float32),
    )(x, expert_w, expert_b, comb)


def kernel(x, gate_w, gate_b, expert_w, expert_b):
    top_idx, w, comb = _gating(x, gate_w, gate_b)
    final = _dense_experts(x, expert_w, expert_b, comb)
    return final, top_idx


# speedup vs baseline: 1.3492x; 1.3492x over previous
import functools
import jax
import jax.numpy as jnp
from jax.experimental import pallas as pl
from jax.experimental.pallas import tpu as pltpu

N_TOK = 4096
D_MODEL = 2048
N_EXP = 8
TOPK = 2


def _gating_body(x_ref, gw_ref, gb_ref, idx_ref, w_ref, comb_ref):
    x = x_ref[...]
    logits = jax.lax.dot_general(
        x, gw_ref[...], (((1,), (1,)), ((), ())),
        preferred_element_type=jnp.float32) + gb_ref[...]
    e_iota = jax.lax.broadcasted_iota(jnp.int32, logits.shape, 1)
    m1 = jnp.max(logits, axis=1, keepdims=True)
    i1 = jnp.min(jnp.where(logits == m1, e_iota, N_EXP), axis=1, keepdims=True)
    l2 = jnp.where(e_iota == i1, -jnp.inf, logits)
    m2 = jnp.max(l2, axis=1, keepdims=True)
    i2 = jnp.min(jnp.where(l2 == m2, e_iota, N_EXP), axis=1, keepdims=True)
    w0 = 1.0 / (1.0 + jnp.exp(m2 - m1))
    w1 = 1.0 - w0
    idx_ref[...] = jnp.concatenate([i1, i2], axis=1)
    w_ref[...] = jnp.concatenate([w0, w1], axis=1)
    comb_ref[...] = (jnp.where(e_iota == i1, w0, 0.0)
                     + jnp.where(e_iota == i2, w1, 0.0))


def _gating(x, gate_w, gate_b):
    blk = 512
    grid = N_TOK // blk
    return pl.pallas_call(
        _gating_body,
        grid=(grid,),
        in_specs=[
            pl.BlockSpec((blk, D_MODEL), lambda t: (t, 0)),
            pl.BlockSpec((N_EXP, D_MODEL), lambda t: (0, 0)),
            pl.BlockSpec((1, N_EXP), lambda t: (0, 0)),
        ],
        out_specs=[
            pl.BlockSpec((blk, TOPK), lambda t: (t, 0)),
            pl.BlockSpec((blk, TOPK), lambda t: (t, 0)),
            pl.BlockSpec((blk, N_EXP), lambda t: (t, 0)),
        ],
        out_shape=[
            jax.ShapeDtypeStruct((N_TOK, TOPK), jnp.int32),
            jax.ShapeDtypeStruct((N_TOK, TOPK), jnp.float32),
            jax.ShapeDtypeStruct((N_TOK, N_EXP), jnp.float32),
        ],
    )(x, gate_w, gate_b.reshape(1, N_EXP))


def _dense_body(x_ref, w_ref, b_ref, comb_ref, out_ref):
    e = pl.program_id(1)
    y = jax.lax.dot_general(
        x_ref[...], w_ref[0], (((1,), (1,)), ((), ())),
        preferred_element_type=jnp.float32)
    comb = comb_ref[...]
    lane = jax.lax.broadcasted_iota(jnp.int32, comb.shape, 1)
    c = jnp.sum(jnp.where(lane == e, comb, 0.0), axis=1, keepdims=True)
    contrib = (y + b_ref[0]) * c

    @pl.when(e == 0)
    def _():
        out_ref[...] = contrib

    @pl.when(e > 0)
    def _():
        out_ref[...] = out_ref[...] + contrib


def _dense_experts(x, expert_w, expert_b, comb):
    tblk = 512
    grid = (N_TOK // tblk, N_EXP)
    return pl.pallas_call(
        _dense_body,
        grid=grid,
        in_specs=[
            pl.BlockSpec((tblk, D_MODEL), lambda t, e: (t, 0)),
            pl.BlockSpec((1, D_MODEL, D_MODEL), lambda t, e: (e, 0, 0)),
            pl.BlockSpec((1, 1, D_MODEL), lambda t, e: (e, 0, 0)),
            pl.BlockSpec((tblk, N_EXP), lambda t, e: (t, 0)),
        ],
        out_specs=pl.BlockSpec((tblk, D_MODEL), lambda t, e: (t, 0)),
        out_shape=jax.ShapeDtypeStruct((N_TOK, D_MODEL), jnp.float32),
    )(x, expert_w, expert_b.reshape(N_EXP, 1, D_MODEL), comb)


def kernel(x, gate_w, gate_b, expert_w, expert_b):
    top_idx, w, comb = _gating(x, gate_w, gate_b)
    final = _dense_experts(x, expert_w, expert_b, comb)
    return final, top_idx
